# baseline (speedup 1.0000x reference)
import jax
import jax.numpy as jnp
from jax.experimental import pallas as pl
from jax.experimental.pallas import tpu as pltpu

_B = 4
_L = 8192
_H = 896
_H4 = 224
_K = 6
_BL = 1024
_NS = 16
_JB = _L // (2 * _BL)


def _score_half(h, w1_ref, b1_ref, w2b, b2):
    x = jnp.dot(h, w1_ref[...], preferred_element_type=jnp.float32)
    x = jnp.maximum(x + b1_ref[...], 0.0)
    xb = x.astype(jnp.bfloat16)
    s = jax.lax.dot_general(w2b, xb, (((0,), (1,)), ((), ())),
                            preferred_element_type=jnp.float32)
    return s + b2


def _body(b2_ref, ha_ref, hb_ref, w1_ref, b1_ref, w2_ref,
          s_ref, idx_ref, mask_ref, acc_ref):
    i = pl.program_id(0)
    w2b = w2_ref[...].astype(jnp.bfloat16)
    sa = _score_half(ha_ref[0], w1_ref, b1_ref, w2b, b2_ref[0])
    sb = _score_half(hb_ref[0], w1_ref, b1_ref, w2b, b2_ref[0])
    s_ref[0, 0, 0:_BL] = sa[0]
    s_ref[0, 0, _BL:2 * _BL] = sb[0]
    row = pl.ds(i // _JB, 1)
    base = (i % _JB) * 2 * _BL
    acc_ref[row, pl.ds(base, _BL)] = sa
    acc_ref[row, pl.ds(base + _BL, _BL)] = sb

    @pl.when(i == _NS - 1)
    def _epilogue():
        cur = acc_ref[...]
        col = jax.lax.broadcasted_iota(jnp.int32, (_B, _L), 1)
        lane = jax.lax.broadcasted_iota(jnp.int32, (_B, 128), 1)
        mask_acc = jnp.zeros((_B, _L), jnp.float32)
        idx_acc = jnp.zeros((_B, 128), jnp.int32)
        for k in range(_K):
            m = jnp.max(cur, axis=1, keepdims=True)
            idx = jnp.min(jnp.where(cur == m, col, _L), axis=1, keepdims=True)
            onehot = col == idx
            mask_acc = jnp.where(onehot, 1.0, mask_acc)
            cur = jnp.where(onehot, -jnp.inf, cur)
            idx_acc = jnp.where(lane == k, idx, idx_acc)
        mask_ref[...] = mask_acc
        idx_ref[...] = idx_acc


@jax.jit
def kernel(hidden_states, W1, b1, W2, b2):
    b, l, h = hidden_states.shape
    h3 = hidden_states.reshape(2 * _NS, _BL, _H)
    scores, idx128, mask = pl.pallas_call(
        _body,
        grid=(_NS,),
        in_specs=[
            pl.BlockSpec(memory_space=pltpu.SMEM),
            pl.BlockSpec((1, _BL, _H), lambda i: (2 * i, 0, 0)),
            pl.BlockSpec((1, _BL, _H), lambda i: (2 * i + 1, 0, 0)),
            pl.BlockSpec((_H, _H4), lambda i: (0, 0)),
            pl.BlockSpec((1, _H4), lambda i: (0, 0)),
            pl.BlockSpec((_H4, 1), lambda i: (0, 0)),
        ],
        out_specs=(
            pl.BlockSpec((1, 1, 2 * _BL), lambda i: (i, 0, 0)),
            pl.BlockSpec((_B, 128), lambda i: (0, 0)),
            pl.BlockSpec((_B, _L), lambda i: (0, 0)),
        ),
        out_shape=(
            jax.ShapeDtypeStruct((_NS, 1, 2 * _BL), jnp.float32),
            jax.ShapeDtypeStruct((_B, 128), jnp.int32),
            jax.ShapeDtypeStruct((_B, _L), jnp.float32),
        ),
        scratch_shapes=[pltpu.VMEM((_B, _L), jnp.float32)],
        compiler_params=pltpu.CompilerParams(
            dimension_semantics=("arbitrary",)),
    )(b2, h3, h3, W1.astype(jnp.bfloat16), b1.reshape(1, _H4), W2)
    return scores.reshape(b, l), idx128[:, :_K], mask
